# Initial kernel scaffold; baseline (speedup 1.0000x reference)
import jax
import jax.numpy as jnp
from jax.experimental import pallas as pl


def kernel(feat_in, nn_list, bin_ids, weight):
    raise NotImplementedError("write your pallas kernel here")


# speedup vs baseline: 15.1386x; 15.1386x over previous
import functools

import jax
import jax.numpy as jnp
from jax import lax
from jax.experimental import pallas as pl
from jax.experimental.pallas import tpu as pltpu
from jax.experimental.pallas import tpu_sc as plsc

SPATIAL = 27
CH = 128
K = 32
NW = 32
C_NODES = 8
EDGE_ROWS = (C_NODES * K) // 128
ACC_ROWS = C_NODES * SPATIAL


def _sc_bin_accumulate(n_pad, chunks_per_worker):
    n_chunks = n_pad // C_NODES
    mesh = plsc.VectorSubcoreMesh(core_axis_name="c", subcore_axis_name="s")

    @functools.partial(
        pl.kernel,
        out_type=jax.ShapeDtypeStruct((n_pad * SPATIAL, CH), jnp.float32),
        mesh=mesh,
        scratch_types=[
            pltpu.VMEM((EDGE_ROWS, 128), jnp.int32),
            pltpu.VMEM((EDGE_ROWS, 128), jnp.int32),
            pltpu.VMEM((EDGE_ROWS, 128), jnp.int32),
            pltpu.VMEM((EDGE_ROWS, 128, CH), jnp.float32),
            pltpu.VMEM((ACC_ROWS, CH), jnp.float32),
            pltpu.VMEM_SHARED((16 * ACC_ROWS, CH), jnp.float32),
        ],
    )
    def sc_kernel(feat_hbm, nn_hbm, bin_hbm, acc_hbm,
                  idx_v, bin_v, dst_v, rows_v, zeros_v, acc_sh):
        sid = lax.axis_index("s")
        wid = sid * 2 + lax.axis_index("c")
        acc_base = sid * ACC_ROWS

        @pl.loop(0, ACC_ROWS)
        def _zinit(r):
            for q in range(CH // 16):
                zeros_v[r, pl.ds(q * 16, 16)] = jnp.zeros((16,), jnp.float32)

        @pl.loop(0, chunks_per_worker)
        def _chunk(t):
            c = wid * chunks_per_worker + t
            pltpu.sync_copy(nn_hbm.at[c], idx_v)
            pltpu.sync_copy(bin_hbm.at[c], bin_v)
            for j in range(EDGE_ROWS):
                pltpu.sync_copy(feat_hbm.at[idx_v.at[j]], rows_v.at[j])

            pltpu.sync_copy(zeros_v, acc_sh.at[pl.ds(acc_base, ACC_ROWS)])

            for j in range(EDGE_ROWS):
                for i in range(8):
                    node_local = (j * 128 + i * 16) // K
                    dst_v[j, pl.ds(i * 16, 16)] = (
                        bin_v[j, pl.ds(i * 16, 16)]
                        + (acc_base + node_local * SPATIAL))

            for j in range(EDGE_ROWS):
                pltpu.sync_copy(rows_v.at[j], acc_sh.at[dst_v.at[j]], add=True)

            pltpu.sync_copy(acc_sh.at[pl.ds(acc_base, ACC_ROWS)],
                            acc_hbm.at[pl.ds(c * ACC_ROWS, ACC_ROWS)])

    return sc_kernel


def _tc_matmul_body(a_ref, w_ref, o_ref):
    o_ref[...] = jnp.dot(a_ref[...], w_ref[...],
                         preferred_element_type=jnp.float32)


def _tc_matmul(acc2, w2, bm):
    n_pad = acc2.shape[0]
    kdim = SPATIAL * CH
    return pl.pallas_call(
        _tc_matmul_body,
        grid=(n_pad // bm,),
        in_specs=[
            pl.BlockSpec((bm, kdim), lambda i: (i, 0)),
            pl.BlockSpec((kdim, CH), lambda i: (0, 0)),
        ],
        out_specs=pl.BlockSpec((bm, CH), lambda i: (i, 0)),
        out_shape=jax.ShapeDtypeStruct((n_pad, CH), jnp.float32),
    )(acc2, w2)


def kernel(feat_in, nn_list, bin_ids, weight):
    n, k = nn_list.shape
    assert k == K and feat_in.shape[1] == CH

    nodes_per_worker_chunked = -(-n // (NW * C_NODES))
    chunks_per_worker = nodes_per_worker_chunked
    n_pad = NW * chunks_per_worker * C_NODES
    n_chunks = n_pad // C_NODES

    nn_flat = nn_list.reshape(-1)
    bin_flat = bin_ids.reshape(-1)
    pad_e = n_pad * K - n * K
    if pad_e:
        nn_flat = jnp.concatenate([nn_flat, jnp.zeros((pad_e,), jnp.int32)])
        bin_flat = jnp.concatenate([bin_flat, jnp.zeros((pad_e,), jnp.int32)])
    nn3 = nn_flat.reshape(n_chunks, EDGE_ROWS, 128)
    bin3 = bin_flat.reshape(n_chunks, EDGE_ROWS, 128)

    acc = _sc_bin_accumulate(n_pad, chunks_per_worker)(feat_in, nn3, bin3)
    acc2 = acc.reshape(n_pad, SPATIAL * CH)

    w2 = weight.reshape(SPATIAL, CH, CH).transpose(0, 2, 1).reshape(
        SPATIAL * CH, CH)

    out = _tc_matmul(acc2, w2, bm=512)
    return out[:n].reshape(n, CH, 1)
